# baseline (speedup 1.0000x reference)
import jax
import jax.numpy as jnp
from jax.experimental import pallas as pl

HCH = 96
NGR = 256
NBLK = 2000
EBLK = 8000


def _lrelu(v):
    return jnp.where(v >= 0, v, 0.01 * v)


def _elu(v):
    return jnp.where(v > 0, v, jnp.exp(jnp.minimum(v, 0.0)) - 1.0)


def _gru_math(xg, h, wih_t, whh_t, bih, bhh):
    gi = jnp.dot(xg, wih_t, preferred_element_type=jnp.float32) + bih
    gh = jnp.dot(h, whh_t, preferred_element_type=jnp.float32) + bhh
    r = jax.nn.sigmoid(gi[:, :HCH] + gh[:, :HCH])
    z = jax.nn.sigmoid(gi[:, HCH:2 * HCH] + gh[:, HCH:2 * HCH])
    n = jnp.tanh(gi[:, 2 * HCH:] + r * gh[:, 2 * HCH:])
    return (1.0 - z) * n + z * h


def _node_in_body(x_ref, w_ref, b_ref, ar_ref, o_ref, a_ref):
    x0 = _lrelu(jnp.dot(x_ref[...], w_ref[...], preferred_element_type=jnp.float32) + b_ref[...])
    o_ref[...] = x0
    a_ref[...] = jnp.dot(x0, ar_ref[...], preferred_element_type=jnp.float32)


def _node_in(x, w_t, b, ar_col):
    n, d = x.shape
    return pl.pallas_call(
        _node_in_body,
        grid=(n // NBLK,),
        in_specs=[
            pl.BlockSpec((NBLK, d), lambda i: (i, 0)),
            pl.BlockSpec(w_t.shape, lambda i: (0, 0)),
            pl.BlockSpec(b.shape, lambda i: (0, 0)),
            pl.BlockSpec(ar_col.shape, lambda i: (0, 0)),
        ],
        out_specs=(
            pl.BlockSpec((NBLK, HCH), lambda i: (i, 0)),
            pl.BlockSpec((NBLK, 1), lambda i: (i, 0)),
        ),
        out_shape=(
            jax.ShapeDtypeStruct((n, HCH), jnp.float32),
            jax.ShapeDtypeStruct((n, 1), jnp.float32),
        ),
    )(x, w_t, b, ar_col)


def _gate_edge_body(xj_ref, ea_ref, w1a_ref, w1b_ref, attl_ref, w2_ref, al_ref, m_ref):
    xj = xj_ref[...]
    hj = _lrelu(jnp.dot(xj, w1a_ref[...], preferred_element_type=jnp.float32)
                + jnp.dot(ea_ref[...], w1b_ref[...], preferred_element_type=jnp.float32))
    al_ref[...] = jnp.dot(hj, attl_ref[...], preferred_element_type=jnp.float32)
    m_ref[...] = jnp.dot(xj, w2_ref[...], preferred_element_type=jnp.float32)


def _gate_edge(xj, ea, w1a, w1b, attl_col, w2_t):
    e = xj.shape[0]
    return pl.pallas_call(
        _gate_edge_body,
        grid=(e // EBLK,),
        in_specs=[
            pl.BlockSpec((EBLK, HCH), lambda i: (i, 0)),
            pl.BlockSpec((EBLK, ea.shape[1]), lambda i: (i, 0)),
            pl.BlockSpec(w1a.shape, lambda i: (0, 0)),
            pl.BlockSpec(w1b.shape, lambda i: (0, 0)),
            pl.BlockSpec(attl_col.shape, lambda i: (0, 0)),
            pl.BlockSpec(w2_t.shape, lambda i: (0, 0)),
        ],
        out_specs=(
            pl.BlockSpec((EBLK, 1), lambda i: (i, 0)),
            pl.BlockSpec((EBLK, HCH), lambda i: (i, 0)),
        ),
        out_shape=(
            jax.ShapeDtypeStruct((e, 1), jnp.float32),
            jax.ShapeDtypeStruct((e, HCH), jnp.float32),
        ),
    )(xj, ea, w1a, w1b, attl_col, w2_t)


def _gat_node_body(x_ref, w_ref, asrc_ref, adst_ref, xp_ref, s_ref, d_ref):
    xp = jnp.dot(x_ref[...], w_ref[...], preferred_element_type=jnp.float32)
    xp_ref[...] = xp
    s_ref[...] = jnp.dot(xp, asrc_ref[...], preferred_element_type=jnp.float32)
    d_ref[...] = jnp.dot(xp, adst_ref[...], preferred_element_type=jnp.float32)


def _gat_node(x, w_t, asrc_col, adst_col):
    n = x.shape[0]
    return pl.pallas_call(
        _gat_node_body,
        grid=(n // NBLK,),
        in_specs=[
            pl.BlockSpec((NBLK, HCH), lambda i: (i, 0)),
            pl.BlockSpec(w_t.shape, lambda i: (0, 0)),
            pl.BlockSpec(asrc_col.shape, lambda i: (0, 0)),
            pl.BlockSpec(adst_col.shape, lambda i: (0, 0)),
        ],
        out_specs=(
            pl.BlockSpec((NBLK, HCH), lambda i: (i, 0)),
            pl.BlockSpec((NBLK, 1), lambda i: (i, 0)),
            pl.BlockSpec((NBLK, 1), lambda i: (i, 0)),
        ),
        out_shape=(
            jax.ShapeDtypeStruct((n, HCH), jnp.float32),
            jax.ShapeDtypeStruct((n, 1), jnp.float32),
            jax.ShapeDtypeStruct((n, 1), jnp.float32),
        ),
    )(x, w_t, asrc_col, adst_col)


def _gru_body(g_ref, h_ref, bias_ref, wih_ref, whh_ref, bih_ref, bhh_ref, o_ref):
    xg = _elu(g_ref[...] + bias_ref[...])
    h = h_ref[...]
    o_ref[...] = jax.nn.relu(
        _gru_math(xg, h, wih_ref[...], whh_ref[...], bih_ref[...], bhh_ref[...]))


def _gru_call(conv, h, bias, gp):
    n = conv.shape[0]
    wih_t = gp["wih"].T
    whh_t = gp["whh"].T
    bih = gp["bih"][None, :]
    bhh = gp["bhh"][None, :]
    return pl.pallas_call(
        _gru_body,
        grid=(n // NBLK,),
        in_specs=[
            pl.BlockSpec((NBLK, HCH), lambda i: (i, 0)),
            pl.BlockSpec((NBLK, HCH), lambda i: (i, 0)),
            pl.BlockSpec(bias.shape, lambda i: (0, 0)),
            pl.BlockSpec(wih_t.shape, lambda i: (0, 0)),
            pl.BlockSpec(whh_t.shape, lambda i: (0, 0)),
            pl.BlockSpec(bih.shape, lambda i: (0, 0)),
            pl.BlockSpec(bhh.shape, lambda i: (0, 0)),
        ],
        out_specs=pl.BlockSpec((NBLK, HCH), lambda i: (i, 0)),
        out_shape=jax.ShapeDtypeStruct((n, HCH), jnp.float32),
    )(conv, h, bias, wih_t, whh_t, bih, bhh)


def _mol_head_body(*refs):
    s = refs[0:3]
    outs = []
    for t in range(3):
        lin_t, bias, wih_t, whh_t, bih, bhh, l2t, l2b = refs[3 + 8 * t: 11 + 8 * t]
        out = jax.nn.relu(s[t][...])
        for _ in range(2):
            h = _elu(
                jnp.dot(out, lin_t[...], preferred_element_type=jnp.float32) + bias[...])
            out = jax.nn.relu(
                _gru_math(h, out, wih_t[...], whh_t[...], bih[...], bhh[...]))
        outs.append(jnp.dot(out, l2t[...], preferred_element_type=jnp.float32) + l2b[...])
    w1t, b1, w2t, b2, w3t, b3 = refs[27:33]
    o_ref = refs[33]
    xc = jnp.concatenate(outs, axis=-1)
    h1 = jax.nn.relu(jnp.dot(xc, w1t[...], preferred_element_type=jnp.float32) + b1[...])
    h2 = jax.nn.relu(jnp.dot(h1, w2t[...], preferred_element_type=jnp.float32) + b2[...])
    o_ref[...] = jnp.dot(h2, w3t[...], preferred_element_type=jnp.float32) + b3[...]


def _seg_softmax(s, seg, num):
    m = jax.ops.segment_max(s, seg, num_segments=num)
    m = jnp.where(jnp.isfinite(m), m, 0.0)
    e = jnp.exp(s - m[seg])
    d = jax.ops.segment_sum(e, seg, num_segments=num)
    return e / (d[seg] + 1e-16)


def _tower(x, ei, ea, batch, p):
    n = x.shape[0]
    src, dst = ei[0], ei[1]
    g = p["gate"]
    x0, ar = _node_in(x, p["lin1_w"].T, p["lin1_b"][None, :], g["att_r"].T)

    w1t = g["lin1_w"].T
    al, m = _gate_edge(x0[src], ea, w1t[:HCH], w1t[HCH:], g["att_l"].T, g["lin2_w"].T)
    s = _lrelu(al[:, 0] + ar[dst, 0])
    a = _seg_softmax(s, dst, n)
    conv = jax.ops.segment_sum(m * a[:, None], dst, num_segments=n)
    xs = _gru_call(conv, x0, g["bias"][None, :], p["gru"])

    for layer in p["atom"]:
        c = layer["conv"]
        xp, asr, adr = _gat_node(xs, c["lin_w"].T, c["att_src"][:, None], c["att_dst"][:, None])
        s = _lrelu(asr[src, 0] + adr[dst, 0])
        a = _seg_softmax(s, dst, n)
        conv = jax.ops.segment_sum(xp[src] * a[:, None], dst, num_segments=n)
        xs = _gru_call(conv, xs, c["bias"][None, :], layer["gru"])

    return jax.ops.segment_sum(xs, batch, num_segments=NGR)


@jax.jit
def kernel(x1, edge_index1, edge_attr1, batch1, x2, edge_index2, edge_attr2, batch2, x3, edge_index3, edge_attr3, batch3, p1, p2, p3, bind):
    s1 = _tower(x1, edge_index1, edge_attr1, batch1, p1)
    s2 = _tower(x2, edge_index2, edge_attr2, batch2, p2)
    s3 = _tower(x3, edge_index3, edge_attr3, batch3, p3)
    args = [s1, s2, s3]
    for p in (p1, p2, p3):
        mc, mg = p["mol_conv"], p["mol_gru"]
        args += [mc["lin_w"].T, mc["bias"][None, :], mg["wih"].T, mg["whh"].T,
                 mg["bih"][None, :], mg["bhh"][None, :],
                 p["lin2_w"].T, p["lin2_b"][None, :]]
    args += [bind["w1"].T, bind["b1"][None, :], bind["w2"].T, bind["b2"][None, :],
             bind["w3"].T, bind["b3"][None, :]]
    return pl.pallas_call(
        _mol_head_body,
        out_shape=jax.ShapeDtypeStruct((NGR, 3), jnp.float32),
    )(*args)


# speedup vs baseline: 1.0011x; 1.0011x over previous
import jax
import jax.numpy as jnp
from jax.experimental import pallas as pl

HCH = 96
NGR = 256
NBLK = 2000
EBLK = 8000


def _lrelu(v):
    return jnp.where(v >= 0, v, 0.01 * v)


def _elu(v):
    return jnp.where(v > 0, v, jnp.exp(jnp.minimum(v, 0.0)) - 1.0)


def _gru_math(xg, h, wih_t, whh_t, bih, bhh):
    gi = jnp.dot(xg, wih_t, preferred_element_type=jnp.float32) + bih
    gh = jnp.dot(h, whh_t, preferred_element_type=jnp.float32) + bhh
    r = jax.nn.sigmoid(gi[:, :HCH] + gh[:, :HCH])
    z = jax.nn.sigmoid(gi[:, HCH:2 * HCH] + gh[:, HCH:2 * HCH])
    n = jnp.tanh(gi[:, 2 * HCH:] + r * gh[:, 2 * HCH:])
    return (1.0 - z) * n + z * h


def _node_in_body(x_ref, w_ref, b_ref, ar_ref, o_ref, a_ref):
    x0 = _lrelu(jnp.dot(x_ref[...], w_ref[...], preferred_element_type=jnp.float32) + b_ref[...])
    o_ref[...] = x0
    a_ref[...] = jnp.dot(x0, ar_ref[...], preferred_element_type=jnp.float32)


def _node_in(x, w_t, b, ar_col):
    n, d = x.shape
    return pl.pallas_call(
        _node_in_body,
        grid=(n // NBLK,),
        in_specs=[
            pl.BlockSpec((NBLK, d), lambda i: (i, 0)),
            pl.BlockSpec(w_t.shape, lambda i: (0, 0)),
            pl.BlockSpec(b.shape, lambda i: (0, 0)),
            pl.BlockSpec(ar_col.shape, lambda i: (0, 0)),
        ],
        out_specs=(
            pl.BlockSpec((NBLK, HCH), lambda i: (i, 0)),
            pl.BlockSpec((NBLK, 1), lambda i: (i, 0)),
        ),
        out_shape=(
            jax.ShapeDtypeStruct((n, HCH), jnp.float32),
            jax.ShapeDtypeStruct((n, 1), jnp.float32),
        ),
    )(x, w_t, b, ar_col)


def _gate_edge_body(xj_ref, ea_ref, w1a_ref, w1b_ref, attl_ref, w2_ref, al_ref, m_ref):
    xj = xj_ref[...]
    hj = _lrelu(jnp.dot(xj, w1a_ref[...], preferred_element_type=jnp.float32)
                + jnp.dot(ea_ref[...], w1b_ref[...], preferred_element_type=jnp.float32))
    al_ref[...] = jnp.dot(hj, attl_ref[...], preferred_element_type=jnp.float32)
    m_ref[...] = jnp.dot(xj, w2_ref[...], preferred_element_type=jnp.float32)


def _gate_edge(xj, ea, w1a, w1b, attl_col, w2_t):
    e = xj.shape[0]
    return pl.pallas_call(
        _gate_edge_body,
        grid=(e // EBLK,),
        in_specs=[
            pl.BlockSpec((EBLK, HCH), lambda i: (i, 0)),
            pl.BlockSpec((EBLK, ea.shape[1]), lambda i: (i, 0)),
            pl.BlockSpec(w1a.shape, lambda i: (0, 0)),
            pl.BlockSpec(w1b.shape, lambda i: (0, 0)),
            pl.BlockSpec(attl_col.shape, lambda i: (0, 0)),
            pl.BlockSpec(w2_t.shape, lambda i: (0, 0)),
        ],
        out_specs=(
            pl.BlockSpec((EBLK, 1), lambda i: (i, 0)),
            pl.BlockSpec((EBLK, HCH), lambda i: (i, 0)),
        ),
        out_shape=(
            jax.ShapeDtypeStruct((e, 1), jnp.float32),
            jax.ShapeDtypeStruct((e, HCH), jnp.float32),
        ),
    )(xj, ea, w1a, w1b, attl_col, w2_t)


def _gat_node_body(x_ref, w_ref, asrc_ref, adst_ref, xp_ref, s_ref, d_ref):
    xp = jnp.dot(x_ref[...], w_ref[...], preferred_element_type=jnp.float32)
    xp_ref[...] = xp
    s_ref[...] = jnp.dot(xp, asrc_ref[...], preferred_element_type=jnp.float32)
    d_ref[...] = jnp.dot(xp, adst_ref[...], preferred_element_type=jnp.float32)


def _gat_node(x, w_t, asrc_col, adst_col):
    n = x.shape[0]
    return pl.pallas_call(
        _gat_node_body,
        grid=(n // NBLK,),
        in_specs=[
            pl.BlockSpec((NBLK, HCH), lambda i: (i, 0)),
            pl.BlockSpec(w_t.shape, lambda i: (0, 0)),
            pl.BlockSpec(asrc_col.shape, lambda i: (0, 0)),
            pl.BlockSpec(adst_col.shape, lambda i: (0, 0)),
        ],
        out_specs=(
            pl.BlockSpec((NBLK, HCH), lambda i: (i, 0)),
            pl.BlockSpec((NBLK, 1), lambda i: (i, 0)),
            pl.BlockSpec((NBLK, 1), lambda i: (i, 0)),
        ),
        out_shape=(
            jax.ShapeDtypeStruct((n, HCH), jnp.float32),
            jax.ShapeDtypeStruct((n, 1), jnp.float32),
            jax.ShapeDtypeStruct((n, 1), jnp.float32),
        ),
    )(x, w_t, asrc_col, adst_col)


def _gru_body(g_ref, h_ref, bias_ref, wih_ref, whh_ref, bih_ref, bhh_ref, o_ref):
    xg = _elu(g_ref[...] + bias_ref[...])
    h = h_ref[...]
    o_ref[...] = jax.nn.relu(
        _gru_math(xg, h, wih_ref[...], whh_ref[...], bih_ref[...], bhh_ref[...]))


def _gru_gat_body(g_ref, h_ref, bias_ref, wih_ref, whh_ref, bih_ref, bhh_ref,
                  w_ref, asrc_ref, adst_ref, xs_ref, xp_ref, s_ref, d_ref):
    xg = _elu(g_ref[...] + bias_ref[...])
    xs = jax.nn.relu(
        _gru_math(xg, h_ref[...], wih_ref[...], whh_ref[...], bih_ref[...], bhh_ref[...]))
    xs_ref[...] = xs
    xp = jnp.dot(xs, w_ref[...], preferred_element_type=jnp.float32)
    xp_ref[...] = xp
    s_ref[...] = jnp.dot(xp, asrc_ref[...], preferred_element_type=jnp.float32)
    d_ref[...] = jnp.dot(xp, adst_ref[...], preferred_element_type=jnp.float32)


def _gru_gat(conv, h, bias, gp, w_t, asrc_col, adst_col):
    n = conv.shape[0]
    wih_t = gp["wih"].T
    whh_t = gp["whh"].T
    bih = gp["bih"][None, :]
    bhh = gp["bhh"][None, :]
    full = lambda a: pl.BlockSpec(a.shape, lambda i: (0, 0))
    blk = lambda w: pl.BlockSpec((NBLK, w), lambda i: (i, 0))
    return pl.pallas_call(
        _gru_gat_body,
        grid=(n // NBLK,),
        in_specs=[
            blk(HCH), blk(HCH), full(bias), full(wih_t), full(whh_t),
            full(bih), full(bhh), full(w_t), full(asrc_col), full(adst_col),
        ],
        out_specs=(blk(HCH), blk(HCH), blk(1), blk(1)),
        out_shape=(
            jax.ShapeDtypeStruct((n, HCH), jnp.float32),
            jax.ShapeDtypeStruct((n, HCH), jnp.float32),
            jax.ShapeDtypeStruct((n, 1), jnp.float32),
            jax.ShapeDtypeStruct((n, 1), jnp.float32),
        ),
    )(conv, h, bias, wih_t, whh_t, bih, bhh, w_t, asrc_col, adst_col)


def _gru_call(conv, h, bias, gp):
    n = conv.shape[0]
    wih_t = gp["wih"].T
    whh_t = gp["whh"].T
    bih = gp["bih"][None, :]
    bhh = gp["bhh"][None, :]
    return pl.pallas_call(
        _gru_body,
        grid=(n // NBLK,),
        in_specs=[
            pl.BlockSpec((NBLK, HCH), lambda i: (i, 0)),
            pl.BlockSpec((NBLK, HCH), lambda i: (i, 0)),
            pl.BlockSpec(bias.shape, lambda i: (0, 0)),
            pl.BlockSpec(wih_t.shape, lambda i: (0, 0)),
            pl.BlockSpec(whh_t.shape, lambda i: (0, 0)),
            pl.BlockSpec(bih.shape, lambda i: (0, 0)),
            pl.BlockSpec(bhh.shape, lambda i: (0, 0)),
        ],
        out_specs=pl.BlockSpec((NBLK, HCH), lambda i: (i, 0)),
        out_shape=jax.ShapeDtypeStruct((n, HCH), jnp.float32),
    )(conv, h, bias, wih_t, whh_t, bih, bhh)


def _mol_head_body(*refs):
    s = refs[0:3]
    outs = []
    for t in range(3):
        lin_t, bias, wih_t, whh_t, bih, bhh, l2t, l2b = refs[3 + 8 * t: 11 + 8 * t]
        out = jax.nn.relu(s[t][...])
        for _ in range(2):
            h = _elu(
                jnp.dot(out, lin_t[...], preferred_element_type=jnp.float32) + bias[...])
            out = jax.nn.relu(
                _gru_math(h, out, wih_t[...], whh_t[...], bih[...], bhh[...]))
        outs.append(jnp.dot(out, l2t[...], preferred_element_type=jnp.float32) + l2b[...])
    w1t, b1, w2t, b2, w3t, b3 = refs[27:33]
    o_ref = refs[33]
    xc = jnp.concatenate(outs, axis=-1)
    h1 = jax.nn.relu(jnp.dot(xc, w1t[...], preferred_element_type=jnp.float32) + b1[...])
    h2 = jax.nn.relu(jnp.dot(h1, w2t[...], preferred_element_type=jnp.float32) + b2[...])
    o_ref[...] = jnp.dot(h2, w3t[...], preferred_element_type=jnp.float32) + b3[...]


def _seg_softmax(s, seg, num):
    m = jax.ops.segment_max(s, seg, num_segments=num)
    m = jnp.where(jnp.isfinite(m), m, 0.0)
    e = jnp.exp(s - m[seg])
    d = jax.ops.segment_sum(e, seg, num_segments=num)
    return e / (d[seg] + 1e-16)


def _tower(x, ei, ea, batch, p):
    n = x.shape[0]
    src, dst = ei[0], ei[1]
    g = p["gate"]
    x0, ar = _node_in(x, p["lin1_w"].T, p["lin1_b"][None, :], g["att_r"].T)

    w1t = g["lin1_w"].T
    al, m = _gate_edge(x0[src], ea, w1t[:HCH], w1t[HCH:], g["att_l"].T, g["lin2_w"].T)
    s = _lrelu(al[:, 0] + ar[dst, 0])
    a = _seg_softmax(s, dst, n)
    conv = jax.ops.segment_sum(m * a[:, None], dst, num_segments=n)

    bias = g["bias"][None, :]
    gp = p["gru"]
    h_state = x0
    for layer in p["atom"]:
        c = layer["conv"]
        xs, xp, asr, adr = _gru_gat(conv, h_state, bias, gp, c["lin_w"].T,
                                    c["att_src"][:, None], c["att_dst"][:, None])
        s = _lrelu(asr[src, 0] + adr[dst, 0])
        a = _seg_softmax(s, dst, n)
        conv = jax.ops.segment_sum(xp[src] * a[:, None], dst, num_segments=n)
        h_state = xs
        bias = c["bias"][None, :]
        gp = layer["gru"]
    xs = _gru_call(conv, h_state, bias, gp)
    return jax.ops.segment_sum(xs, batch, num_segments=NGR)


@jax.jit
def kernel(x1, edge_index1, edge_attr1, batch1, x2, edge_index2, edge_attr2, batch2, x3, edge_index3, edge_attr3, batch3, p1, p2, p3, bind):
    s1 = _tower(x1, edge_index1, edge_attr1, batch1, p1)
    s2 = _tower(x2, edge_index2, edge_attr2, batch2, p2)
    s3 = _tower(x3, edge_index3, edge_attr3, batch3, p3)
    args = [s1, s2, s3]
    for p in (p1, p2, p3):
        mc, mg = p["mol_conv"], p["mol_gru"]
        args += [mc["lin_w"].T, mc["bias"][None, :], mg["wih"].T, mg["whh"].T,
                 mg["bih"][None, :], mg["bhh"][None, :],
                 p["lin2_w"].T, p["lin2_b"][None, :]]
    args += [bind["w1"].T, bind["b1"][None, :], bind["w2"].T, bind["b2"][None, :],
             bind["w3"].T, bind["b3"][None, :]]
    return pl.pallas_call(
        _mol_head_body,
        out_shape=jax.ShapeDtypeStruct((NGR, 3), jnp.float32),
    )(*args)
